# baseline (speedup 1.0000x reference)
import functools

import jax
import jax.numpy as jnp
from jax import lax
from jax.experimental import pallas as pl
from jax.experimental.pallas import tpu as pltpu
from jax.experimental.pallas import tpu_sc as plsc

RANK = 32
BATCH = 16384

_INFO = plsc.get_sparse_core_info()
_NC = _INFO.num_cores
_NS = _INFO.num_subcores
_NW = _NC * _NS
_BPW = BATCH // _NW


def _sc_gather_body(tidx_hbm, ridx_hbm, cidx_hbm, tt_hbm, ut_hbm, it_hbm,
                    out_t, out_u, out_i,
                    ti_v, ri_v, ci_v, sem):
    wid = lax.axis_index("s") * _NC + lax.axis_index("c")
    base = wid * _BPW
    sl = pl.ds(base, _BPW)
    pltpu.sync_copy(tidx_hbm.at[sl], ti_v)
    pltpu.sync_copy(ridx_hbm.at[sl], ri_v)
    pltpu.sync_copy(cidx_hbm.at[sl], ci_v)

    def group(g, carry):
        vt = ti_v[pl.ds(g * 16, 16)]
        vr = ri_v[pl.ds(g * 16, 16)]
        vc = ci_v[pl.ds(g * 16, 16)]
        for k in range(16):
            r = base + g * 16 + k
            pltpu.async_copy(tt_hbm.at[pl.ds(vt[k], 1)], out_t.at[pl.ds(r, 1)], sem)
            pltpu.async_copy(ut_hbm.at[pl.ds(vr[k], 1)], out_u.at[pl.ds(r, 1)], sem)
            pltpu.async_copy(it_hbm.at[pl.ds(vc[k], 1)], out_i.at[pl.ds(r, 1)], sem)
        return carry

    lax.fori_loop(0, _BPW // 16, group, 0)
    pltpu.make_async_copy(tt_hbm.at[pl.ds(0, _BPW)], out_t.at[sl], sem).wait()
    pltpu.make_async_copy(ut_hbm.at[pl.ds(0, _BPW)], out_u.at[sl], sem).wait()
    pltpu.make_async_copy(it_hbm.at[pl.ds(0, _BPW)], out_i.at[sl], sem).wait()


@jax.jit
def _sc_gather(tIdx, rIdx, cIdx, time_table, user_table, item_table):
    mesh = plsc.VectorSubcoreMesh(core_axis_name="c", subcore_axis_name="s")
    f = functools.partial(
        pl.kernel,
        mesh=mesh,
        out_type=(
            jax.ShapeDtypeStruct((BATCH, RANK), jnp.float32),
            jax.ShapeDtypeStruct((BATCH, RANK), jnp.float32),
            jax.ShapeDtypeStruct((BATCH, RANK), jnp.float32),
        ),
        scratch_types=[
            pltpu.VMEM((_BPW,), jnp.int32),
            pltpu.VMEM((_BPW,), jnp.int32),
            pltpu.VMEM((_BPW,), jnp.int32),
            pltpu.SemaphoreType.DMA,
        ],
    )(_sc_gather_body)
    return f(tIdx, rIdx, cIdx, time_table, user_table, item_table)


def _tc_body(et_ref, eu_ref, ei_ref, wt_ref, wu_ref, wi_ref,
             bt_ref, bu_ref, bi_ref, o_ref):
    t = jnp.dot(et_ref[...], wt_ref[...], preferred_element_type=jnp.float32) + bt_ref[...]
    u = jnp.dot(eu_ref[...], wu_ref[...], preferred_element_type=jnp.float32) + bu_ref[...]
    i = jnp.dot(ei_ref[...], wi_ref[...], preferred_element_type=jnp.float32) + bi_ref[...]
    o_ref[...] = jnp.sum(t * u * i, axis=-1)


_TC_BLOCK = 2048


@jax.jit
def _tc_combine(et, eu, ei, WtT, WuT, WiT, bt, bu, bi):
    grid = BATCH // _TC_BLOCK
    emb_spec = pl.BlockSpec((_TC_BLOCK, RANK), lambda i: (i, 0))
    w_spec = pl.BlockSpec((RANK, RANK), lambda i: (0, 0))
    b_spec = pl.BlockSpec((1, RANK), lambda i: (0, 0))
    return pl.pallas_call(
        _tc_body,
        grid=(grid,),
        in_specs=[emb_spec, emb_spec, emb_spec, w_spec, w_spec, w_spec,
                  b_spec, b_spec, b_spec],
        out_specs=pl.BlockSpec((_TC_BLOCK,), lambda i: (i,)),
        out_shape=jax.ShapeDtypeStruct((BATCH,), jnp.float32),
    )(et, eu, ei, WtT, WuT, WiT, bt, bu, bi)


def kernel(tIdx, rIdx, cIdx, time_table, user_table, item_table,
           Wt, bt, Wu, bu, Wi, bi):
    et, eu, ei = _sc_gather(tIdx, rIdx, cIdx, time_table, user_table, item_table)
    return _tc_combine(et, eu, ei, Wt.T, Wu.T, Wi.T,
                       bt.reshape(1, RANK), bu.reshape(1, RANK),
                       bi.reshape(1, RANK))


# speedup vs baseline: 2.8987x; 2.8987x over previous
import functools

import jax
import jax.numpy as jnp
from jax import lax
from jax.experimental import pallas as pl
from jax.experimental.pallas import tpu as pltpu
from jax.experimental.pallas import tpu_sc as plsc

RANK = 32
BATCH = 16384

_INFO = plsc.get_sparse_core_info()
_NC = _INFO.num_cores
_NS = _INFO.num_subcores
_NW = _NC * _NS
_BPW = BATCH // _NW


def _sc_gather_body(tidx_hbm, ridx_hbm, cidx_hbm, tt_hbm, ut_hbm, it_hbm,
                    out_t, out_u, out_i,
                    ti_v, ri_v, ci_v, tr_v, ur_v, ir_v, sem):
    wid = lax.axis_index("s") * _NC + lax.axis_index("c")
    base = wid * _BPW
    sl = pl.ds(base, _BPW)
    pltpu.sync_copy(tidx_hbm.at[sl], ti_v)
    pltpu.sync_copy(ridx_hbm.at[sl], ri_v)
    pltpu.sync_copy(cidx_hbm.at[sl], ci_v)

    def chunk(c, carry):
        for k in range(4):
            o = c * 64 + k * 16
            vt = ti_v[pl.ds(o, 16)]
            vr = ri_v[pl.ds(o, 16)]
            vc = ci_v[pl.ds(o, 16)]
            for l in range(16):
                r = k * 16 + l
                pltpu.async_copy(tt_hbm.at[pl.ds(vt[l], 1)], tr_v.at[pl.ds(r, 1)], sem)
                pltpu.async_copy(ut_hbm.at[pl.ds(vr[l], 1)], ur_v.at[pl.ds(r, 1)], sem)
                pltpu.async_copy(it_hbm.at[pl.ds(vc[l], 1)], ir_v.at[pl.ds(r, 1)], sem)
        pltpu.make_async_copy(tt_hbm.at[pl.ds(0, 64)], tr_v, sem).wait()
        pltpu.make_async_copy(ut_hbm.at[pl.ds(0, 64)], ur_v, sem).wait()
        pltpu.make_async_copy(it_hbm.at[pl.ds(0, 64)], ir_v, sem).wait()
        oo = pl.ds(base + c * 64, 64)
        pltpu.sync_copy(tr_v, out_t.at[oo])
        pltpu.sync_copy(ur_v, out_u.at[oo])
        pltpu.sync_copy(ir_v, out_i.at[oo])
        return carry

    lax.fori_loop(0, _BPW // 64, chunk, 0)


@jax.jit
def _sc_gather(tIdx, rIdx, cIdx, time_table, user_table, item_table):
    mesh = plsc.VectorSubcoreMesh(core_axis_name="c", subcore_axis_name="s")
    f = functools.partial(
        pl.kernel,
        mesh=mesh,
        out_type=(
            jax.ShapeDtypeStruct((BATCH, RANK), jnp.float32),
            jax.ShapeDtypeStruct((BATCH, RANK), jnp.float32),
            jax.ShapeDtypeStruct((BATCH, RANK), jnp.float32),
        ),
        scratch_types=[
            pltpu.VMEM((_BPW,), jnp.int32),
            pltpu.VMEM((_BPW,), jnp.int32),
            pltpu.VMEM((_BPW,), jnp.int32),
            pltpu.VMEM((64, RANK), jnp.float32),
            pltpu.VMEM((64, RANK), jnp.float32),
            pltpu.VMEM((64, RANK), jnp.float32),
            pltpu.SemaphoreType.DMA,
        ],
    )(_sc_gather_body)
    return f(tIdx, rIdx, cIdx, time_table, user_table, item_table)


def _tc_body(et_ref, eu_ref, ei_ref, wt_ref, wu_ref, wi_ref,
             bt_ref, bu_ref, bi_ref, o_ref):
    t = jnp.dot(et_ref[...], wt_ref[...], preferred_element_type=jnp.float32) + bt_ref[...]
    u = jnp.dot(eu_ref[...], wu_ref[...], preferred_element_type=jnp.float32) + bu_ref[...]
    i = jnp.dot(ei_ref[...], wi_ref[...], preferred_element_type=jnp.float32) + bi_ref[...]
    o_ref[...] = jnp.sum(t * u * i, axis=-1)


_TC_BLOCK = 2048


@jax.jit
def _tc_combine(et, eu, ei, WtT, WuT, WiT, bt, bu, bi):
    grid = BATCH // _TC_BLOCK
    emb_spec = pl.BlockSpec((_TC_BLOCK, RANK), lambda i: (i, 0))
    w_spec = pl.BlockSpec((RANK, RANK), lambda i: (0, 0))
    b_spec = pl.BlockSpec((1, RANK), lambda i: (0, 0))
    return pl.pallas_call(
        _tc_body,
        grid=(grid,),
        in_specs=[emb_spec, emb_spec, emb_spec, w_spec, w_spec, w_spec,
                  b_spec, b_spec, b_spec],
        out_specs=pl.BlockSpec((_TC_BLOCK,), lambda i: (i,)),
        out_shape=jax.ShapeDtypeStruct((BATCH,), jnp.float32),
    )(et, eu, ei, WtT, WuT, WiT, bt, bu, bi)


def kernel(tIdx, rIdx, cIdx, time_table, user_table, item_table,
           Wt, bt, Wu, bu, Wi, bi):
    et, eu, ei = _sc_gather(tIdx, rIdx, cIdx, time_table, user_table, item_table)
    return _tc_combine(et, eu, ei, Wt.T, Wu.T, Wi.T,
                       bt.reshape(1, RANK), bu.reshape(1, RANK),
                       bi.reshape(1, RANK))
